# Initial kernel scaffold; baseline (speedup 1.0000x reference)
import jax
import jax.numpy as jnp
from jax.experimental import pallas as pl


def kernel(special_ids, special_mask, table):
    raise NotImplementedError("write your pallas kernel here")


# speedup vs baseline: 2.3641x; 2.3641x over previous
import functools
import jax
import jax.numpy as jnp
from jax import lax
from jax.experimental import pallas as pl
from jax.experimental.pallas import tpu as pltpu
from jax.experimental.pallas import tpu_sc as plsc

B = 16384
L = 50
D = 64
NC = 2
NS = 16
NW = NC * NS
ROWS_PER_W = B // NW
CHUNK = 16
NCHUNK = ROWS_PER_W // CHUNK

_mesh = plsc.VectorSubcoreMesh(core_axis_name="c", subcore_axis_name="s")

_GATHER_DNUMS = lax.GatherDimensionNumbers(
    offset_dims=(), collapsed_slice_dims=(0,), start_index_map=(0,))


def _bcast_lane0(vec):
    idx = jnp.zeros((16, 1), jnp.int32)
    return lax.gather(vec, idx, _GATHER_DNUMS, slice_sizes=(1,),
                      mode=lax.GatherScatterMode.PROMISE_IN_BOUNDS)


@functools.partial(
    pl.kernel,
    mesh=_mesh,
    out_type=jax.ShapeDtypeStruct((B, D), jnp.float32),
    scratch_types=[
        pltpu.VMEM((CHUNK, L), jnp.int32),
        pltpu.VMEM((CHUNK * L + 16,), jnp.int32),
        pltpu.VMEM((CHUNK, L, D), jnp.float32),
        pltpu.VMEM((CHUNK, D), jnp.float32),
        pltpu.SemaphoreType.DMA,
    ],
    compiler_params=pltpu.CompilerParams(use_tc_tiling_on_sc=False),
)
def _pooled_lookup(ids_hbm, mask_hbm, table_hbm, out_hbm,
                   ids_v, mask_v, rows_v, out_v, sem):
    wid = lax.axis_index("s") * NC + lax.axis_index("c")
    base_row = wid * ROWS_PER_W

    def chunk_body(ci, carry):
        row0 = base_row + ci * CHUNK
        pltpu.sync_copy(ids_hbm.at[pl.ds(row0, CHUNK), :], ids_v)
        pltpu.sync_copy(mask_hbm.at[pl.ds(row0 * L, CHUNK * L)],
                        mask_v.at[pl.ds(0, CHUNK * L)])

        copies = []
        for r in range(CHUNK):
            c = pltpu.make_async_copy(table_hbm.at[ids_v.at[r]], rows_v.at[r], sem)
            c.start()
            copies.append(c)
        for c in copies:
            c.wait()

        def r_body(r, carry):
            def l_body(l, acc):
                a0, a1, a2, a3, cnt = acc
                mwin = mask_v[pl.ds(r * L + l, 16)]
                mf = _bcast_lane0(mwin).astype(jnp.float32)
                a0 = a0 + rows_v[r, l, pl.ds(0, 16)] * mf
                a1 = a1 + rows_v[r, l, pl.ds(16, 16)] * mf
                a2 = a2 + rows_v[r, l, pl.ds(32, 16)] * mf
                a3 = a3 + rows_v[r, l, pl.ds(48, 16)] * mf
                return a0, a1, a2, a3, cnt + mf

            z = jnp.zeros((16,), jnp.float32)
            a0, a1, a2, a3, cnt = lax.fori_loop(0, L, l_body, (z, z, z, z, z))
            inv = 1.0 / jnp.maximum(cnt, 1.0)
            out_v[r, pl.ds(0, 16)] = a0 * inv
            out_v[r, pl.ds(16, 16)] = a1 * inv
            out_v[r, pl.ds(32, 16)] = a2 * inv
            out_v[r, pl.ds(48, 16)] = a3 * inv
            return carry

        lax.fori_loop(0, CHUNK, r_body, 0)

        pltpu.sync_copy(out_v, out_hbm.at[pl.ds(row0, CHUNK), :])
        return carry

    lax.fori_loop(0, NCHUNK, chunk_body, 0)


def kernel(special_ids, special_mask, table):
    return _pooled_lookup(special_ids, special_mask.reshape(-1), table)


# speedup vs baseline: 2.5875x; 1.0945x over previous
import functools
import jax
import jax.numpy as jnp
from jax import lax
from jax.experimental import pallas as pl
from jax.experimental.pallas import tpu as pltpu
from jax.experimental.pallas import tpu_sc as plsc

B = 16384
L = 50
D = 64
NC = 2
NS = 16
NW = NC * NS
ROWS_PER_W = B // NW
CHUNK = 16
NCHUNK = ROWS_PER_W // CHUNK

_mesh = plsc.VectorSubcoreMesh(core_axis_name="c", subcore_axis_name="s")

_GATHER_DNUMS = lax.GatherDimensionNumbers(
    offset_dims=(), collapsed_slice_dims=(0,), start_index_map=(0,))


def _bcast_lane0(vec):
    idx = jnp.zeros((16, 1), jnp.int32)
    return lax.gather(vec, idx, _GATHER_DNUMS, slice_sizes=(1,),
                      mode=lax.GatherScatterMode.PROMISE_IN_BOUNDS)


@functools.partial(
    pl.kernel,
    mesh=_mesh,
    out_type=jax.ShapeDtypeStruct((B, D), jnp.float32),
    scratch_types=[
        pltpu.VMEM((CHUNK, L), jnp.int32),
        pltpu.VMEM((CHUNK, L), jnp.int32),
        pltpu.VMEM((CHUNK * L + 16,), jnp.int32),
        pltpu.VMEM((CHUNK * L + 16,), jnp.int32),
        pltpu.VMEM((CHUNK, L, D), jnp.float32),
        pltpu.VMEM((CHUNK, L, D), jnp.float32),
        pltpu.VMEM((CHUNK, D), jnp.float32),
        pltpu.SemaphoreType.DMA,
        pltpu.SemaphoreType.DMA,
    ],
    compiler_params=pltpu.CompilerParams(use_tc_tiling_on_sc=False),
)
def _pooled_lookup(ids_hbm, mask_hbm, table_hbm, out_hbm,
                   ids_a, ids_b, mask_a, mask_b, rows_a, rows_b,
                   out_v, sem_a, sem_b):
    wid = lax.axis_index("s") * NC + lax.axis_index("c")
    base_row = wid * ROWS_PER_W

    def stage(ci, ids_v, mask_v):
        row0 = base_row + ci * CHUNK
        pltpu.sync_copy(ids_hbm.at[pl.ds(row0, CHUNK), :], ids_v)
        pltpu.sync_copy(mask_hbm.at[pl.ds(row0 * L, CHUNK * L)],
                        mask_v.at[pl.ds(0, CHUNK * L)])

    def fire(ids_v, rows_v, sem):
        for r in range(CHUNK):
            pltpu.make_async_copy(
                table_hbm.at[ids_v.at[r]], rows_v.at[r], sem).start()

    def drain(ids_v, rows_v, sem):
        for r in range(CHUNK):
            pltpu.make_async_copy(
                table_hbm.at[ids_v.at[r]], rows_v.at[r], sem).wait()

    def compute(ci, mask_v, rows_v):
        row0 = base_row + ci * CHUNK

        def r_body(r, carry):
            def l_body(l, acc):
                a0, a1, a2, a3, cnt = acc
                mwin = mask_v[pl.ds(r * L + l, 16)]
                mf = _bcast_lane0(mwin).astype(jnp.float32)
                a0 = a0 + rows_v[r, l, pl.ds(0, 16)] * mf
                a1 = a1 + rows_v[r, l, pl.ds(16, 16)] * mf
                a2 = a2 + rows_v[r, l, pl.ds(32, 16)] * mf
                a3 = a3 + rows_v[r, l, pl.ds(48, 16)] * mf
                return a0, a1, a2, a3, cnt + mf

            z = jnp.zeros((16,), jnp.float32)
            a0, a1, a2, a3, cnt = lax.fori_loop(0, L, l_body, (z, z, z, z, z))
            inv = 1.0 / jnp.maximum(cnt, 1.0)
            out_v[r, pl.ds(0, 16)] = a0 * inv
            out_v[r, pl.ds(16, 16)] = a1 * inv
            out_v[r, pl.ds(32, 16)] = a2 * inv
            out_v[r, pl.ds(48, 16)] = a3 * inv
            return carry

        lax.fori_loop(0, CHUNK, r_body, 0)
        pltpu.sync_copy(out_v, out_hbm.at[pl.ds(row0, CHUNK), :])

    stage(0, ids_a, mask_a)
    fire(ids_a, rows_a, sem_a)

    def k_body(k, carry):
        stage(2 * k + 1, ids_b, mask_b)
        fire(ids_b, rows_b, sem_b)
        drain(ids_a, rows_a, sem_a)
        compute(2 * k, mask_a, rows_a)

        @pl.when(2 * k + 2 < NCHUNK)
        def _():
            stage(2 * k + 2, ids_a, mask_a)
            fire(ids_a, rows_a, sem_a)

        drain(ids_b, rows_b, sem_b)
        compute(2 * k + 1, mask_b, rows_b)
        return carry

    lax.fori_loop(0, NCHUNK // 2, k_body, 0)


def kernel(special_ids, special_mask, table):
    return _pooled_lookup(special_ids, special_mask.reshape(-1), table)


# speedup vs baseline: 2.6983x; 1.0429x over previous
import functools
import jax
import jax.numpy as jnp
from jax import lax
from jax.experimental import pallas as pl
from jax.experimental.pallas import tpu as pltpu
from jax.experimental.pallas import tpu_sc as plsc

B = 16384
L = 50
D = 64
NC = 2
NS = 16
NW = NC * NS
ROWS_PER_W = B // NW
CHUNK = 16
NCHUNK = ROWS_PER_W // CHUNK

_mesh = plsc.VectorSubcoreMesh(core_axis_name="c", subcore_axis_name="s")


@functools.partial(
    pl.kernel,
    mesh=_mesh,
    out_type=jax.ShapeDtypeStruct((B, D), jnp.float32),
    scratch_types=[
        pltpu.VMEM((CHUNK, L), jnp.int32),
        pltpu.VMEM((CHUNK, L), jnp.int32),
        pltpu.VMEM((CHUNK * L + 16,), jnp.int32),
        pltpu.VMEM((CHUNK * L + 16,), jnp.int32),
        pltpu.VMEM((CHUNK, L, D), jnp.float32),
        pltpu.VMEM((CHUNK, L, D), jnp.float32),
        pltpu.VMEM((CHUNK, D), jnp.float32),
        pltpu.SemaphoreType.DMA,
        pltpu.SemaphoreType.DMA,
    ],
    compiler_params=pltpu.CompilerParams(use_tc_tiling_on_sc=False),
)
def _pooled_lookup(ids_hbm, mask_hbm, table_hbm, out_hbm,
                   ids_a, ids_b, mask_a, mask_b, rows_a, rows_b,
                   out_v, sem_a, sem_b):
    wid = lax.axis_index("s") * NC + lax.axis_index("c")
    base_row = wid * ROWS_PER_W

    def stage(ci, ids_v, mask_v):
        row0 = base_row + ci * CHUNK
        pltpu.sync_copy(ids_hbm.at[pl.ds(row0, CHUNK), :], ids_v)
        pltpu.sync_copy(mask_hbm.at[pl.ds(row0 * L, CHUNK * L)],
                        mask_v.at[pl.ds(0, CHUNK * L)])

    def fire(ids_v, rows_v, sem):
        for r in range(CHUNK):
            pltpu.make_async_copy(
                table_hbm.at[ids_v.at[r]], rows_v.at[r], sem).start()

    def drain(ids_v, rows_v, sem):
        for r in range(CHUNK):
            pltpu.make_async_copy(
                table_hbm.at[ids_v.at[r]], rows_v.at[r], sem).wait()

    def compute(ci, mask_v, rows_v):
        row0 = base_row + ci * CHUNK

        def r_body(r, carry):
            wf = [mask_v[pl.ds(r * L + 16 * j, 16)].astype(jnp.float32)
                  for j in range(4)]
            z = jnp.zeros((16,), jnp.float32)
            acc = [z, z, z, z]
            cnt = z
            for l in range(L):
                mf = jnp.full((16,), wf[l // 16][l % 16])
                cnt = cnt + mf
                for d in range(4):
                    acc[d] = acc[d] + rows_v[r, l, pl.ds(16 * d, 16)] * mf
            inv = 1.0 / jnp.maximum(cnt, 1.0)
            for d in range(4):
                out_v[r, pl.ds(16 * d, 16)] = acc[d] * inv
            return carry

        lax.fori_loop(0, CHUNK, r_body, 0)
        pltpu.sync_copy(out_v, out_hbm.at[pl.ds(row0, CHUNK), :])

    stage(0, ids_a, mask_a)
    fire(ids_a, rows_a, sem_a)

    def k_body(k, carry):
        stage(2 * k + 1, ids_b, mask_b)
        fire(ids_b, rows_b, sem_b)
        drain(ids_a, rows_a, sem_a)
        compute(2 * k, mask_a, rows_a)

        @pl.when(2 * k + 2 < NCHUNK)
        def _():
            stage(2 * k + 2, ids_a, mask_a)
            fire(ids_a, rows_a, sem_a)

        drain(ids_b, rows_b, sem_b)
        compute(2 * k + 1, mask_b, rows_b)
        return carry

    lax.fori_loop(0, NCHUNK // 2, k_body, 0)


def kernel(special_ids, special_mask, table):
    return _pooled_lookup(special_ids, special_mask.reshape(-1), table)


# speedup vs baseline: 2.7203x; 1.0081x over previous
import functools
import jax
import jax.numpy as jnp
from jax import lax
from jax.experimental import pallas as pl
from jax.experimental.pallas import tpu as pltpu
from jax.experimental.pallas import tpu_sc as plsc

B = 16384
L = 50
D = 64
NC = 2
NS = 16
NW = NC * NS
ROWS_PER_W = B // NW
CHUNK = 16
NCHUNK = ROWS_PER_W // CHUNK

_mesh = plsc.VectorSubcoreMesh(core_axis_name="c", subcore_axis_name="s")


@functools.partial(
    pl.kernel,
    mesh=_mesh,
    out_type=jax.ShapeDtypeStruct((B, D), jnp.float32),
    scratch_types=[
        pltpu.VMEM((CHUNK * L,), jnp.int32),
        pltpu.VMEM((CHUNK * L,), jnp.int32),
        pltpu.VMEM((CHUNK * L + 16,), jnp.int32),
        pltpu.VMEM((CHUNK * L + 16,), jnp.int32),
        pltpu.VMEM((CHUNK * L, D), jnp.float32),
        pltpu.VMEM((CHUNK * L, D), jnp.float32),
        pltpu.VMEM((CHUNK, D), jnp.float32),
        pltpu.SemaphoreType.DMA,
        pltpu.SemaphoreType.DMA,
    ],
    compiler_params=pltpu.CompilerParams(use_tc_tiling_on_sc=False),
)
def _pooled_lookup(ids_hbm, mask_hbm, table_hbm, out_hbm,
                   ids_a, ids_b, mask_a, mask_b, rows_a, rows_b,
                   out_v, sem_a, sem_b):
    wid = lax.axis_index("s") * NC + lax.axis_index("c")
    base_row = wid * ROWS_PER_W

    def stage(ci, ids_v, mask_v):
        row0 = base_row + ci * CHUNK
        pltpu.sync_copy(ids_hbm.at[pl.ds(row0 * L, CHUNK * L)], ids_v)
        pltpu.sync_copy(mask_hbm.at[pl.ds(row0 * L, CHUNK * L)],
                        mask_v.at[pl.ds(0, CHUNK * L)])

    def fire(ids_v, rows_v, sem):
        pltpu.make_async_copy(table_hbm.at[ids_v], rows_v, sem).start()

    def drain(ids_v, rows_v, sem):
        pltpu.make_async_copy(table_hbm.at[ids_v], rows_v, sem).wait()

    def compute(ci, mask_v, rows_v):
        row0 = base_row + ci * CHUNK

        def r_body(r, carry):
            wf = [mask_v[pl.ds(r * L + 16 * j, 16)].astype(jnp.float32)
                  for j in range(4)]
            z = jnp.zeros((16,), jnp.float32)
            acc = [z, z, z, z]
            cnt = z
            for l in range(L):
                mf = jnp.full((16,), wf[l // 16][l % 16])
                cnt = cnt + mf
                for d in range(4):
                    acc[d] = acc[d] + rows_v[r * L + l, pl.ds(16 * d, 16)] * mf
            inv = 1.0 / jnp.maximum(cnt, 1.0)
            for d in range(4):
                out_v[r, pl.ds(16 * d, 16)] = acc[d] * inv
            return carry

        lax.fori_loop(0, CHUNK, r_body, 0)
        pltpu.sync_copy(out_v, out_hbm.at[pl.ds(row0, CHUNK), :])

    stage(0, ids_a, mask_a)
    fire(ids_a, rows_a, sem_a)

    def k_body(k, carry):
        stage(2 * k + 1, ids_b, mask_b)
        fire(ids_b, rows_b, sem_b)
        drain(ids_a, rows_a, sem_a)
        compute(2 * k, mask_a, rows_a)

        @pl.when(2 * k + 2 < NCHUNK)
        def _():
            stage(2 * k + 2, ids_a, mask_a)
            fire(ids_a, rows_a, sem_a)

        drain(ids_b, rows_b, sem_b)
        compute(2 * k + 1, mask_b, rows_b)
        return carry

    lax.fori_loop(0, NCHUNK // 2, k_body, 0)


def kernel(special_ids, special_mask, table):
    return _pooled_lookup(special_ids.reshape(-1), special_mask.reshape(-1),
                          table)
